# Initial kernel scaffold; baseline (speedup 1.0000x reference)
import jax
import jax.numpy as jnp
from jax.experimental import pallas as pl


def kernel(x, adj_t, W1, b1, W2, b2):
    raise NotImplementedError("write your pallas kernel here")


# speedup vs baseline: 1.0206x; 1.0206x over previous
import functools

import jax
import jax.numpy as jnp
from jax.experimental import pallas as pl
from jax.experimental.pallas import tpu as pltpu

BM = 400


def _layer1_kernel(a_ref, x_ref, w1_ref, b1_ref, w2_ref, b2_ref,
                   y2_ref, y1_s):
    i = pl.program_id(0)

    @pl.when(i == 0)
    def _():
        y1 = jnp.dot(x_ref[...].astype(jnp.bfloat16),
                     w1_ref[...].astype(jnp.bfloat16),
                     preferred_element_type=jnp.float32) + b1_ref[...]
        y1_s[...] = y1.astype(jnp.bfloat16)

    h = jnp.dot(a_ref[...].astype(jnp.bfloat16), y1_s[...],
                preferred_element_type=jnp.float32)
    h = jnp.maximum(h, 0.0)
    y2 = jnp.dot(h.astype(jnp.bfloat16), w2_ref[...].astype(jnp.bfloat16),
                 preferred_element_type=jnp.float32) + b2_ref[...]
    y2_ref[...] = y2.astype(jnp.bfloat16)


def _layer2_kernel(a_ref, y2_ref, out_ref):
    out_ref[...] = jnp.dot(a_ref[...].astype(jnp.bfloat16), y2_ref[...],
                           preferred_element_type=jnp.float32)


@jax.jit
def kernel(x, adj_t, W1, b1, W2, b2):
    n, d_in = x.shape
    d_hid = W1.shape[1]
    d_out = W2.shape[1]
    nblk = pl.cdiv(n, BM)

    b1r = b1.reshape(1, d_hid)
    b2r = b2.reshape(1, d_out)

    y2 = pl.pallas_call(
        _layer1_kernel,
        grid=(nblk,),
        in_specs=[
            pl.BlockSpec((BM, n), lambda i: (i, 0)),
            pl.BlockSpec((n, d_in), lambda i: (0, 0)),
            pl.BlockSpec((d_in, d_hid), lambda i: (0, 0)),
            pl.BlockSpec((1, d_hid), lambda i: (0, 0)),
            pl.BlockSpec((d_hid, d_out), lambda i: (0, 0)),
            pl.BlockSpec((1, d_out), lambda i: (0, 0)),
        ],
        out_specs=pl.BlockSpec((BM, d_out), lambda i: (i, 0)),
        out_shape=jax.ShapeDtypeStruct((n, d_out), jnp.bfloat16),
        scratch_shapes=[pltpu.VMEM((n, d_hid), jnp.bfloat16)],
        compiler_params=pltpu.CompilerParams(
            dimension_semantics=(pltpu.GridDimensionSemantics.ARBITRARY,),
        ),
    )(adj_t, x, W1, b1r, W2, b2r)

    out = pl.pallas_call(
        _layer2_kernel,
        grid=(nblk,),
        in_specs=[
            pl.BlockSpec((BM, n), lambda i: (i, 0)),
            pl.BlockSpec((n, d_out), lambda i: (0, 0)),
        ],
        out_specs=pl.BlockSpec((BM, d_out), lambda i: (i, 0)),
        out_shape=jax.ShapeDtypeStruct((n, d_out), jnp.float32),
        compiler_params=pltpu.CompilerParams(
            dimension_semantics=(pltpu.GridDimensionSemantics.ARBITRARY,),
        ),
    )(adj_t, y2)

    return out
